# baseline (speedup 1.0000x reference)
import functools

import jax
import jax.numpy as jnp
from jax import lax
from jax.experimental import pallas as pl
from jax.experimental.pallas import tpu as pltpu
from jax.experimental.pallas import tpu_sc as plsc

B, S, D, E, K = 4, 2048, 768, 8, 2
NC, NS, L = 2, 16, 16
NW = NC * NS


def _gate_kernel(x_ref, gw_ref, gb_ref, out_ref, st_ref):
    xb = x_ref[0]
    gw = gw_ref[...]
    lt = lax.dot_general(gw, xb, (((0,), (1,)), ((), ())),
                         preferred_element_type=jnp.float32)
    lt = lt + gb_ref[...]
    out_ref[0] = lt
    m = jnp.max(lt, axis=1, keepdims=True)
    z = jnp.sum(jnp.exp(lt - m), axis=1, keepdims=True)
    st_ref[0] = jnp.concatenate([m, z], axis=1)


def _gate_logits(x, gate_w, gate_b):
    return pl.pallas_call(
        _gate_kernel,
        grid=(B,),
        in_specs=[
            pl.BlockSpec((1, S, D), lambda i: (i, 0, 0)),
            pl.BlockSpec((D, E), lambda i: (0, 0)),
            pl.BlockSpec((E, 1), lambda i: (0, 0)),
        ],
        out_specs=[
            pl.BlockSpec((1, E, S), lambda i: (i, 0, 0)),
            pl.BlockSpec((1, E, 2), lambda i: (i, 0, 0)),
        ],
        out_shape=[
            jax.ShapeDtypeStruct((B, E, S), jnp.float32),
            jax.ShapeDtypeStruct((B, E, 2), jnp.float32),
        ],
    )(x, gate_w, gate_b)


def _sc_body(logits_hbm, x_hbm, idx_out, vals_out, rows_out,
             lrow, iv, sv, gi, rows, sem):
    c = lax.axis_index("c")
    sub = lax.axis_index("s")
    wid = sub * NC + c
    bq = wid // E
    eq = wid - bq * E

    pltpu.sync_copy(logits_hbm.at[pl.ds(wid, 1)], lrow)

    lane = lax.iota(jnp.int32, L)
    neg = jnp.float32(-3.0e38)
    bigi = jnp.int32(1 << 30)

    def body(i, carry):
        m1, i1, m2, i2 = carry
        v = lrow[0, pl.ds(i * L, L)]
        idxs = i * L + lane
        gt1 = v > m1
        gt2 = v > m2
        m2n = jnp.where(gt1, m1, jnp.where(gt2, v, m2))
        i2n = jnp.where(gt1, i1, jnp.where(gt2, idxs, i2))
        m1n = jnp.where(gt1, v, m1)
        i1n = jnp.where(gt1, idxs, i1)
        return m1n, i1n, m2n, i2n

    zf = jnp.zeros((L,), jnp.float32)
    zi = jnp.zeros((L,), jnp.int32)
    m1, i1, m2, i2 = lax.fori_loop(0, S // L, body, (zf + neg, zi, zf + neg, zi))

    gmax = jnp.max(m1)
    gidx = jnp.min(jnp.where(m1 == gmax, i1, bigi))
    hit = jnp.logical_and(m1 == gmax, i1 == gidx)
    m1b = jnp.where(hit, m2, m1)
    i1b = jnp.where(hit, i2, i1)
    g2 = jnp.max(m1b)
    gidx2 = jnp.min(jnp.where(m1b == g2, i1b, bigi))

    row = eq * B + bq
    iv[0] = jnp.where(lane == 0, gidx, jnp.where(lane == 1, gidx2, 0))
    sv[0] = jnp.where(lane == 0, gmax, jnp.where(lane == 1, g2, 0.0))
    pltpu.sync_copy(iv, idx_out.at[pl.ds(row, 1)])
    pltpu.sync_copy(sv, vals_out.at[pl.ds(row, 1)])

    gi[...] = bq * S + jnp.where(lane == 1, gidx2, gidx)
    pltpu.async_copy(x_hbm.at[gi.at[pl.ds(0, K)]], rows, sem).wait()
    pltpu.sync_copy(rows, rows_out.at[pl.ds(row * K, K)])


def _sc_gate_topk(logits2, xflat):
    mesh = plsc.VectorSubcoreMesh(core_axis_name="c", subcore_axis_name="s",
                                  num_cores=NC, num_subcores=NS)
    fn = pl.kernel(
        _sc_body,
        out_type=[
            jax.ShapeDtypeStruct((NW, L), jnp.int32),
            jax.ShapeDtypeStruct((NW, L), jnp.float32),
            jax.ShapeDtypeStruct((NW * K, D), jnp.float32),
        ],
        mesh=mesh,
        compiler_params=pltpu.CompilerParams(needs_layout_passes=False),
        scratch_types=[
            pltpu.VMEM((1, S), jnp.float32),
            pltpu.VMEM((1, L), jnp.int32),
            pltpu.VMEM((1, L), jnp.float32),
            pltpu.VMEM((L,), jnp.int32),
            pltpu.VMEM((K, D), jnp.float32),
            pltpu.SemaphoreType.DMA,
        ],
    )
    return fn(logits2, xflat)


def _fused_kernel(idx_ref, vals_ref, st_ref, bias_ref, rows_ref, w_ref,
                  out_ref, eo_scr):
    bq = pl.program_id(0)

    @pl.when(bq == 0)
    def _():
        for e in range(E):
            eo_scr[e] = lax.dot_general(
                rows_ref[e], w_ref[e], (((1,), (0,)), ((), ())),
                preferred_element_type=jnp.float32)

    out_ref[...] = jnp.zeros((1, S, D), jnp.float32)
    for e in range(E):
        bv = bias_ref[e, 0]
        m = st_ref[bq * E + e, 0]
        z = st_ref[bq * E + e, 1]
        for k in range(K):
            tok = idx_ref[e * B + bq, k]
            v = vals_ref[e * B + bq, k]
            sc = z * jnp.exp(jnp.full((1, D), m - v, jnp.float32))
            out_ref[0, pl.ds(tok, 1), :] += (
                (eo_scr[e, pl.ds(bq * K + k, 1), :] + bv) * sc)


def _expert_scatter(idx_out, vals_out, stats, bias, rows, w):
    return pl.pallas_call(
        _fused_kernel,
        grid=(B,),
        in_specs=[
            pl.BlockSpec(memory_space=pltpu.SMEM),
            pl.BlockSpec(memory_space=pltpu.SMEM),
            pl.BlockSpec(memory_space=pltpu.SMEM),
            pl.BlockSpec(memory_space=pltpu.SMEM),
            pl.BlockSpec((E, B * K, D), lambda i: (0, 0, 0)),
            pl.BlockSpec((E, D, D), lambda i: (0, 0, 0)),
        ],
        out_specs=pl.BlockSpec((1, S, D), lambda i: (i, 0, 0)),
        out_shape=jax.ShapeDtypeStruct((B, S, D), jnp.float32),
        scratch_shapes=[pltpu.VMEM((E, B * K, D), jnp.float32)],
    )(idx_out, vals_out, stats, bias, rows, w)


def kernel(x, gate_w, gate_b, w, b):
    logits, stats = _gate_logits(x, gate_w, gate_b.reshape(E, 1))
    idx_out, vals_out, rows = _sc_gate_topk(
        logits.reshape(NW, S), x.reshape(B * S, D))
    return _expert_scatter(idx_out, vals_out, stats.reshape(NW, 2), b,
                           rows.reshape(E, B * K, D), w)


# speedup vs baseline: 1.0409x; 1.0409x over previous
import functools

import jax
import jax.numpy as jnp
from jax import lax
from jax.experimental import pallas as pl
from jax.experimental.pallas import tpu as pltpu
from jax.experimental.pallas import tpu_sc as plsc

B, S, D, E, K = 4, 2048, 768, 8, 2
NC, NS, L = 2, 16, 16
NW = NC * NS


def _gate_kernel(x_ref, gw_ref, gb_ref, out_ref, st_ref):
    xb = x_ref[0]
    gw = gw_ref[...]
    lt = lax.dot_general(gw, xb, (((0,), (1,)), ((), ())),
                         preferred_element_type=jnp.float32)
    lt = lt + gb_ref[...]
    out_ref[0] = lt
    m = jnp.max(lt, axis=1, keepdims=True)
    z = jnp.sum(jnp.exp(lt - m), axis=1, keepdims=True)
    st_ref[0] = jnp.concatenate([m, z], axis=1)


def _gate_logits(x, gate_w, gate_b):
    return pl.pallas_call(
        _gate_kernel,
        grid=(B,),
        in_specs=[
            pl.BlockSpec((1, S, D), lambda i: (i, 0, 0)),
            pl.BlockSpec((D, E), lambda i: (0, 0)),
            pl.BlockSpec((E, 1), lambda i: (0, 0)),
        ],
        out_specs=[
            pl.BlockSpec((1, E, S), lambda i: (i, 0, 0)),
            pl.BlockSpec((1, E, 2), lambda i: (i, 0, 0)),
        ],
        out_shape=[
            jax.ShapeDtypeStruct((B, E, S), jnp.float32),
            jax.ShapeDtypeStruct((B, E, 2), jnp.float32),
        ],
    )(x, gate_w, gate_b)


def _sc_body(logits_hbm, x_hbm, idx_out, vals_out, rows_out,
             lrow, iv, sv, gi, rows, sem):
    c = lax.axis_index("c")
    sub = lax.axis_index("s")
    wid = sub * NC + c
    bq = wid // E
    eq = wid - bq * E

    pltpu.sync_copy(logits_hbm.at[pl.ds(wid, 1)], lrow)

    lane = lax.iota(jnp.int32, L)
    neg = jnp.float32(-3.0e38)
    bigi = jnp.int32(1 << 30)

    def body(i, carry):
        m1, i1, m2, i2 = carry
        v = lrow[0, pl.ds(i * L, L)]
        idxs = i * L + lane
        gt1 = v > m1
        gt2 = v > m2
        m2n = jnp.where(gt1, m1, jnp.where(gt2, v, m2))
        i2n = jnp.where(gt1, i1, jnp.where(gt2, idxs, i2))
        m1n = jnp.where(gt1, v, m1)
        i1n = jnp.where(gt1, idxs, i1)
        return m1n, i1n, m2n, i2n

    zf = jnp.zeros((L,), jnp.float32)
    zi = jnp.zeros((L,), jnp.int32)
    m1, i1, m2, i2 = lax.fori_loop(0, S // L, body, (zf + neg, zi, zf + neg, zi))

    gmax = jnp.max(m1)
    gidx = jnp.min(jnp.where(m1 == gmax, i1, bigi))
    hit = jnp.logical_and(m1 == gmax, i1 == gidx)
    m1b = jnp.where(hit, m2, m1)
    i1b = jnp.where(hit, i2, i1)
    g2 = jnp.max(m1b)
    gidx2 = jnp.min(jnp.where(m1b == g2, i1b, bigi))

    row = eq * B + bq
    iv[0] = jnp.where(lane == 0, gidx, jnp.where(lane == 1, gidx2, 0))
    sv[0] = jnp.where(lane == 0, gmax, jnp.where(lane == 1, g2, 0.0))
    pltpu.sync_copy(iv, idx_out.at[pl.ds(row, 1)])
    pltpu.sync_copy(sv, vals_out.at[pl.ds(row, 1)])

    gi[...] = bq * S + jnp.where(lane == 1, gidx2, gidx)
    pltpu.async_copy(x_hbm.at[gi.at[pl.ds(0, K)]], rows, sem).wait()
    pltpu.sync_copy(rows, rows_out.at[pl.ds(row * K, K)])


def _sc_gate_topk(logits2, xflat):
    mesh = plsc.VectorSubcoreMesh(core_axis_name="c", subcore_axis_name="s",
                                  num_cores=NC, num_subcores=NS)
    fn = pl.kernel(
        _sc_body,
        out_type=[
            jax.ShapeDtypeStruct((NW, L), jnp.int32),
            jax.ShapeDtypeStruct((NW, L), jnp.float32),
            jax.ShapeDtypeStruct((NW * K, D), jnp.float32),
        ],
        mesh=mesh,
        compiler_params=pltpu.CompilerParams(needs_layout_passes=False),
        scratch_types=[
            pltpu.VMEM((1, S), jnp.float32),
            pltpu.VMEM((1, L), jnp.int32),
            pltpu.VMEM((1, L), jnp.float32),
            pltpu.VMEM((L,), jnp.int32),
            pltpu.VMEM((K, D), jnp.float32),
            pltpu.SemaphoreType.DMA,
        ],
    )
    return fn(logits2, xflat)


def _zero_kernel(out_ref):
    out_ref[...] = jnp.zeros((1, S, D), jnp.float32)


def _zero_fill():
    return pl.pallas_call(
        _zero_kernel,
        grid=(B,),
        out_specs=pl.BlockSpec((1, S, D), lambda i: (i, 0, 0)),
        out_shape=jax.ShapeDtypeStruct((B, S, D), jnp.float32),
    )()


def _scatter_mm_kernel(out0_ref, idx_ref, vals_ref, st_ref, bias_ref,
                       rows_ref, w_ref, out_ref, eo_scr, mrg_scr, sem):
    del out0_ref
    for e in range(E):
        acc = lax.dot_general(rows_ref[e], w_ref[e], (((1,), (0,)), ((), ())),
                              preferred_element_type=jnp.float32)
        eo_scr[:, e * K:(e + 1) * K, :] = acc.reshape(B, K, D)
    subl = lax.broadcasted_iota(jnp.int32, (E * K, 1), 0)
    lanei = lax.broadcasted_iota(jnp.int32, (1, E * K), 1)
    for bb in range(B):
        tokc = jnp.zeros((E * K, 1), jnp.int32)
        tokr = jnp.zeros((1, E * K), jnp.int32)
        mc = jnp.zeros((E * K, 1), jnp.float32)
        zc = jnp.zeros((E * K, 1), jnp.float32)
        vc = jnp.zeros((E * K, 1), jnp.float32)
        bc = jnp.zeros((E * K, 1), jnp.float32)
        for i in range(E * K):
            e, k = i // K, i % K
            t = idx_ref[e * B + bb, k]
            tokc = jnp.where(subl == i, t, tokc)
            tokr = jnp.where(lanei == i, t, tokr)
            mc = jnp.where(subl == i, st_ref[bb * E + e, 0], mc)
            zc = jnp.where(subl == i, st_ref[bb * E + e, 1], zc)
            vc = jnp.where(subl == i, vals_ref[e * B + bb, k], vc)
            bc = jnp.where(subl == i, bias_ref[e, 0], bc)
        scc = zc * jnp.exp(mc - vc)
        scaled = (eo_scr[bb] + bc) * scc
        eq = (tokc == tokr).astype(jnp.float32)
        mrg_scr[bb] = lax.dot_general(eq, scaled, (((1,), (0,)), ((), ())),
                                      precision=lax.Precision.HIGHEST,
                                      preferred_element_type=jnp.float32)
    copies = []
    for bb in range(B):
        for i in range(E * K):
            tok = idx_ref[(i // K) * B + bb, i % K]
            cp = pltpu.make_async_copy(mrg_scr.at[bb, pl.ds(i, 1), :],
                                       out_ref.at[bb, pl.ds(tok, 1), :], sem)
            cp.start()
            copies.append(cp)
    for cp in copies:
        cp.wait()


def _scatter_mm(out0, idx_out, vals_out, stats, bias, rows, w):
    return pl.pallas_call(
        _scatter_mm_kernel,
        in_specs=[
            pl.BlockSpec(memory_space=pltpu.MemorySpace.HBM),
            pl.BlockSpec(memory_space=pltpu.SMEM),
            pl.BlockSpec(memory_space=pltpu.SMEM),
            pl.BlockSpec(memory_space=pltpu.SMEM),
            pl.BlockSpec(memory_space=pltpu.SMEM),
            pl.BlockSpec((E, B * K, D), lambda: (0, 0, 0)),
            pl.BlockSpec((E, D, D), lambda: (0, 0, 0)),
        ],
        out_specs=pl.BlockSpec(memory_space=pltpu.MemorySpace.HBM),
        out_shape=jax.ShapeDtypeStruct((B, S, D), jnp.float32),
        input_output_aliases={0: 0},
        scratch_shapes=[pltpu.VMEM((B, E * K, D), jnp.float32),
                        pltpu.VMEM((B, E * K, D), jnp.float32),
                        pltpu.SemaphoreType.DMA],
    )(out0, idx_out, vals_out, stats, bias, rows, w)


def _fused_kernel(idx_ref, vals_ref, st_ref, bias_ref, rows_ref, w_ref,
                  out_ref, eo_scr):
    bq = pl.program_id(0)

    @pl.when(bq == 0)
    def _():
        for e in range(E):
            eo_scr[e] = lax.dot_general(
                rows_ref[e], w_ref[e], (((1,), (0,)), ((), ())),
                preferred_element_type=jnp.float32)

    out_ref[...] = jnp.zeros((1, S, D), jnp.float32)
    for e in range(E):
        bv = bias_ref[e, 0]
        m = st_ref[bq * E + e, 0]
        z = st_ref[bq * E + e, 1]
        for k in range(K):
            tok = idx_ref[e * B + bq, k]
            v = vals_ref[e * B + bq, k]
            sc = z * jnp.exp(jnp.full((1, D), m - v, jnp.float32))
            out_ref[0, pl.ds(tok, 1), :] += (
                (eo_scr[e, pl.ds(bq * K + k, 1), :] + bv) * sc)


def _expert_scatter(idx_out, vals_out, stats, bias, rows, w):
    return pl.pallas_call(
        _fused_kernel,
        grid=(B,),
        in_specs=[
            pl.BlockSpec(memory_space=pltpu.SMEM),
            pl.BlockSpec(memory_space=pltpu.SMEM),
            pl.BlockSpec(memory_space=pltpu.SMEM),
            pl.BlockSpec(memory_space=pltpu.SMEM),
            pl.BlockSpec((E, B * K, D), lambda i: (0, 0, 0)),
            pl.BlockSpec((E, D, D), lambda i: (0, 0, 0)),
        ],
        out_specs=pl.BlockSpec((1, S, D), lambda i: (i, 0, 0)),
        out_shape=jax.ShapeDtypeStruct((B, S, D), jnp.float32),
        scratch_shapes=[pltpu.VMEM((E, B * K, D), jnp.float32)],
    )(idx_out, vals_out, stats, bias, rows, w)


def kernel(x, gate_w, gate_b, w, b):
    logits, stats = _gate_logits(x, gate_w, gate_b.reshape(E, 1))
    idx_out, vals_out, rows = _sc_gate_topk(
        logits.reshape(NW, S), x.reshape(B * S, D))
    out0 = _zero_fill()
    return _scatter_mm(out0, idx_out, vals_out, stats.reshape(NW, 2), b,
                       rows.reshape(E, B * K, D), w)


# speedup vs baseline: 1.0730x; 1.0309x over previous
import functools

import jax
import jax.numpy as jnp
from jax import lax
from jax.experimental import pallas as pl
from jax.experimental.pallas import tpu as pltpu
from jax.experimental.pallas import tpu_sc as plsc

B, S, D, E, K = 4, 2048, 768, 8, 2
NC, NS, L = 2, 16, 16
NW = NC * NS


def _gate_kernel(x_ref, gw_ref, gb_ref, out_ref, st_ref):
    xb = x_ref[0]
    gw = gw_ref[...]
    lt = lax.dot_general(gw, xb, (((0,), (1,)), ((), ())),
                         preferred_element_type=jnp.float32)
    lt = lt + gb_ref[...]
    out_ref[0] = lt
    m = jnp.max(lt, axis=1, keepdims=True)
    z = jnp.sum(jnp.exp(lt - m), axis=1, keepdims=True)
    st_ref[0] = jnp.concatenate([m, z], axis=1)


def _gate_logits(x, gate_w, gate_b):
    return pl.pallas_call(
        _gate_kernel,
        grid=(B,),
        in_specs=[
            pl.BlockSpec((1, S, D), lambda i: (i, 0, 0)),
            pl.BlockSpec((D, E), lambda i: (0, 0)),
            pl.BlockSpec((E, 1), lambda i: (0, 0)),
        ],
        out_specs=[
            pl.BlockSpec((1, E, S), lambda i: (i, 0, 0)),
            pl.BlockSpec((1, E, 2), lambda i: (i, 0, 0)),
        ],
        out_shape=[
            jax.ShapeDtypeStruct((B, E, S), jnp.float32),
            jax.ShapeDtypeStruct((B, E, 2), jnp.float32),
        ],
    )(x, gate_w, gate_b)


def _sc_body(logits_hbm, x_hbm, idx_out, vals_out, rows_out,
             lrow, iv, sv, gi, rows, sem):
    sub = lax.axis_index("s")
    for j in range(NW // NS):
        _sc_one_pair(sub * (NW // NS) + j, logits_hbm, x_hbm,
                     idx_out, vals_out, rows_out, lrow, iv, sv, gi, rows, sem)


def _sc_one_pair(wid, logits_hbm, x_hbm, idx_out, vals_out, rows_out,
                 lrow, iv, sv, gi, rows, sem):
    bq = wid // E
    eq = wid - bq * E

    pltpu.sync_copy(logits_hbm.at[pl.ds(wid, 1)], lrow)

    lane = lax.iota(jnp.int32, L)
    neg = jnp.float32(-3.0e38)
    bigi = jnp.int32(1 << 30)

    def body(i, carry):
        m1, i1, m2, i2 = carry
        v = lrow[0, pl.ds(i * L, L)]
        idxs = i * L + lane
        gt1 = v > m1
        gt2 = v > m2
        m2n = jnp.where(gt1, m1, jnp.where(gt2, v, m2))
        i2n = jnp.where(gt1, i1, jnp.where(gt2, idxs, i2))
        m1n = jnp.where(gt1, v, m1)
        i1n = jnp.where(gt1, idxs, i1)
        return m1n, i1n, m2n, i2n

    zf = jnp.zeros((L,), jnp.float32)
    zi = jnp.zeros((L,), jnp.int32)
    m1, i1, m2, i2 = lax.fori_loop(0, S // L, body, (zf + neg, zi, zf + neg, zi))

    gmax = jnp.max(m1)
    gidx = jnp.min(jnp.where(m1 == gmax, i1, bigi))
    hit = jnp.logical_and(m1 == gmax, i1 == gidx)
    m1b = jnp.where(hit, m2, m1)
    i1b = jnp.where(hit, i2, i1)
    g2 = jnp.max(m1b)
    gidx2 = jnp.min(jnp.where(m1b == g2, i1b, bigi))

    row = eq * B + bq
    iv[0] = jnp.where(lane == 0, gidx, jnp.where(lane == 1, gidx2, 0))
    sv[0] = jnp.where(lane == 0, gmax, jnp.where(lane == 1, g2, 0.0))
    pltpu.sync_copy(iv, idx_out.at[pl.ds(row, 1)])
    pltpu.sync_copy(sv, vals_out.at[pl.ds(row, 1)])

    gi[...] = bq * S + jnp.where(lane == 1, gidx2, gidx)
    pltpu.async_copy(x_hbm.at[gi.at[pl.ds(0, K)]], rows, sem).wait()
    pltpu.sync_copy(rows, rows_out.at[pl.ds(row * K, K)])


def _sc_gate_topk(logits2, xflat):
    mesh = plsc.VectorSubcoreMesh(core_axis_name="c", subcore_axis_name="s",
                                  num_cores=1, num_subcores=NS)
    fn = pl.kernel(
        _sc_body,
        out_type=[
            jax.ShapeDtypeStruct((NW, L), jnp.int32),
            jax.ShapeDtypeStruct((NW, L), jnp.float32),
            jax.ShapeDtypeStruct((NW * K, D), jnp.float32),
        ],
        mesh=mesh,
        compiler_params=pltpu.CompilerParams(needs_layout_passes=False),
        scratch_types=[
            pltpu.VMEM((1, S), jnp.float32),
            pltpu.VMEM((1, L), jnp.int32),
            pltpu.VMEM((1, L), jnp.float32),
            pltpu.VMEM((L,), jnp.int32),
            pltpu.VMEM((K, D), jnp.float32),
            pltpu.SemaphoreType.DMA,
        ],
    )
    return fn(logits2, xflat)


def _zero_kernel(out_ref):
    out_ref[...] = jnp.zeros((1, S, D), jnp.float32)


def _zero_fill():
    return pl.pallas_call(
        _zero_kernel,
        grid=(B,),
        out_specs=pl.BlockSpec((1, S, D), lambda i: (i, 0, 0)),
        out_shape=jax.ShapeDtypeStruct((B, S, D), jnp.float32),
    )()


def _scatter_mm_kernel(out0_ref, idx_ref, vals_ref, st_ref, bias_ref,
                       rows_ref, w_ref, out_ref, eo_scr, mrg_scr, sem):
    del out0_ref
    for e in range(E):
        acc = lax.dot_general(rows_ref[e], w_ref[e], (((1,), (0,)), ((), ())),
                              preferred_element_type=jnp.float32)
        eo_scr[:, e * K:(e + 1) * K, :] = acc.reshape(B, K, D)
    subl = lax.broadcasted_iota(jnp.int32, (E * K, 1), 0)
    lanei = lax.broadcasted_iota(jnp.int32, (1, E * K), 1)
    for bb in range(B):
        tokc = jnp.zeros((E * K, 1), jnp.int32)
        tokr = jnp.zeros((1, E * K), jnp.int32)
        mc = jnp.zeros((E * K, 1), jnp.float32)
        zc = jnp.zeros((E * K, 1), jnp.float32)
        vc = jnp.zeros((E * K, 1), jnp.float32)
        bc = jnp.zeros((E * K, 1), jnp.float32)
        for i in range(E * K):
            e, k = i // K, i % K
            t = idx_ref[e * B + bb, k]
            tokc = jnp.where(subl == i, t, tokc)
            tokr = jnp.where(lanei == i, t, tokr)
            mc = jnp.where(subl == i, st_ref[bb * E + e, 0], mc)
            zc = jnp.where(subl == i, st_ref[bb * E + e, 1], zc)
            vc = jnp.where(subl == i, vals_ref[e * B + bb, k], vc)
            bc = jnp.where(subl == i, bias_ref[e, 0], bc)
        scc = zc * jnp.exp(mc - vc)
        scaled = (eo_scr[bb] + bc) * scc
        eq = (tokc == tokr).astype(jnp.float32)
        mrg_scr[bb] = lax.dot_general(eq, scaled, (((1,), (0,)), ((), ())),
                                      precision=lax.Precision.HIGHEST,
                                      preferred_element_type=jnp.float32)
    copies = []
    for bb in range(B):
        for i in range(E * K):
            tok = idx_ref[(i // K) * B + bb, i % K]
            cp = pltpu.make_async_copy(mrg_scr.at[bb, pl.ds(i, 1), :],
                                       out_ref.at[bb, pl.ds(tok, 1), :], sem)
            cp.start()
            copies.append(cp)
    for cp in copies:
        cp.wait()


def _scatter_mm(out0, idx_out, vals_out, stats, bias, rows, w):
    return pl.pallas_call(
        _scatter_mm_kernel,
        in_specs=[
            pl.BlockSpec(memory_space=pltpu.MemorySpace.HBM),
            pl.BlockSpec(memory_space=pltpu.SMEM),
            pl.BlockSpec(memory_space=pltpu.SMEM),
            pl.BlockSpec(memory_space=pltpu.SMEM),
            pl.BlockSpec(memory_space=pltpu.SMEM),
            pl.BlockSpec((E, B * K, D), lambda: (0, 0, 0)),
            pl.BlockSpec((E, D, D), lambda: (0, 0, 0)),
        ],
        out_specs=pl.BlockSpec(memory_space=pltpu.MemorySpace.HBM),
        out_shape=jax.ShapeDtypeStruct((B, S, D), jnp.float32),
        input_output_aliases={0: 0},
        scratch_shapes=[pltpu.VMEM((B, E * K, D), jnp.float32),
                        pltpu.VMEM((B, E * K, D), jnp.float32),
                        pltpu.SemaphoreType.DMA],
    )(out0, idx_out, vals_out, stats, bias, rows, w)


def _fused_kernel(idx_ref, vals_ref, st_ref, bias_ref, rows_ref, w_ref,
                  out_ref, eo_scr):
    bq = pl.program_id(0)

    @pl.when(bq == 0)
    def _():
        for e in range(E):
            eo_scr[e] = lax.dot_general(
                rows_ref[e], w_ref[e], (((1,), (0,)), ((), ())),
                preferred_element_type=jnp.float32)

    out_ref[...] = jnp.zeros((1, S, D), jnp.float32)
    for e in range(E):
        bv = bias_ref[e, 0]
        m = st_ref[bq * E + e, 0]
        z = st_ref[bq * E + e, 1]
        for k in range(K):
            tok = idx_ref[e * B + bq, k]
            v = vals_ref[e * B + bq, k]
            sc = z * jnp.exp(jnp.full((1, D), m - v, jnp.float32))
            out_ref[0, pl.ds(tok, 1), :] += (
                (eo_scr[e, pl.ds(bq * K + k, 1), :] + bv) * sc)


def _expert_scatter(idx_out, vals_out, stats, bias, rows, w):
    return pl.pallas_call(
        _fused_kernel,
        grid=(B,),
        in_specs=[
            pl.BlockSpec(memory_space=pltpu.SMEM),
            pl.BlockSpec(memory_space=pltpu.SMEM),
            pl.BlockSpec(memory_space=pltpu.SMEM),
            pl.BlockSpec(memory_space=pltpu.SMEM),
            pl.BlockSpec((E, B * K, D), lambda i: (0, 0, 0)),
            pl.BlockSpec((E, D, D), lambda i: (0, 0, 0)),
        ],
        out_specs=pl.BlockSpec((1, S, D), lambda i: (i, 0, 0)),
        out_shape=jax.ShapeDtypeStruct((B, S, D), jnp.float32),
        scratch_shapes=[pltpu.VMEM((E, B * K, D), jnp.float32)],
    )(idx_out, vals_out, stats, bias, rows, w)


def kernel(x, gate_w, gate_b, w, b):
    logits, stats = _gate_logits(x, gate_w, gate_b.reshape(E, 1))
    idx_out, vals_out, rows = _sc_gate_topk(
        logits.reshape(NW, S), x.reshape(B * S, D))
    out0 = _zero_fill()
    return _scatter_mm(out0, idx_out, vals_out, stats.reshape(NW, 2), b,
                       rows.reshape(E, B * K, D), w)
